# baseline (speedup 1.0000x reference)
import jax
import jax.numpy as jnp
from jax import lax
from jax.experimental import pallas as pl
from jax.experimental.pallas import tpu as pltpu
from jax.experimental.pallas import tpu_sc as plsc

F = 32
B = 524288
RES = (64, 128, 256)
DIMIDS = ((0, 1), (0, 2), (1, 2))
NC, NS = 2, 16
NW = NC * NS
PPW = B // NW
C = 32
NCHUNK = PPW // C
ROW = 4 * F


def _quad_table(plane, R):
    t = plane.transpose(1, 2, 0).reshape(R * R, F)
    n = R * R
    tp = jnp.pad(t, ((0, R + 1), (0, 0)))
    return jnp.concatenate(
        [t, tp[1:n + 1], tp[R:n + R], tp[R + 1:n + R + 1]], axis=1)


def _sc_body(xT, t0, t1, t2, t3, t4, t5, t6, t7, t8, out,
             xv, wb, idxb, gball, outb, semx, semg, semo):
    tabs = (t0, t1, t2, t3, t4, t5, t6, t7, t8)
    wid = lax.axis_index("s") * NC + lax.axis_index("c")
    wbase = wid * PPW

    def fire_x(g, par):
        for d in range(3):
            pltpu.async_copy(
                xT.at[d, pl.ds(wbase + g * C, C)], xv.at[par, d], semx)

    def wait_x(par):
        for d in range(3):
            pltpu.make_async_copy(
                xT.at[d, pl.ds(wbase, C)], xv.at[par, d], semx).wait()

    def comp_idx(par):
        for q in range(C // 16):
            sl = pl.ds(q * 16, 16)
            xd = [xv[par, d, sl] for d in range(3)]
            for l, R in enumerate(RES):
                s = (R - 1) * 0.5
                iv = []
                for d in range(3):
                    gco = (xd[d] + 1.0) * s
                    ic = jnp.maximum(
                        jnp.minimum(gco.astype(jnp.int32), R - 2), 0)
                    wb[par, l, d, sl] = gco - ic.astype(jnp.float32)
                    iv.append(ic)
                idxb[par, 3 * l + 0, sl] = iv[1] * R + iv[0]
                idxb[par, 3 * l + 1, sl] = iv[2] * R + iv[0]
                idxb[par, 3 * l + 2, sl] = iv[2] * R + iv[1]

    def fire_item(l2, par2, slot):
        for i in range(3):
            pltpu.async_copy(
                tabs[3 * l2 + i].at[idxb.at[par2, 3 * l2 + i]],
                gball.at[slot, i], semg)

    def wait_item(slot):
        for i in range(3):
            pltpu.make_async_copy(
                tabs[0].at[idxb.at[0, 0]], gball.at[slot, i], semg).wait()

    def combine(l, par):
        def grp_body(q, c):
            wv = [wb[par, l, d, pl.ds(q * 16, 16)] for d in range(3)]
            cw = []
            for a, b in DIMIDS:
                wx, wy = wv[a], wv[b]
                u = 1.0 - wx
                v = 1.0 - wy
                cw.append((u * v, wx * v, u * wy, wx * wy))
            for p16 in range(16):
                p = q * 16 + p16
                acc = [None, None]
                for i in range(3):
                    w00 = cw[i][0][p16]
                    w01 = cw[i][1][p16]
                    w10 = cw[i][2][p16]
                    w11 = cw[i][3][p16]
                    for h in range(2):
                        r = (w00 * gball[l, i, p, pl.ds(h * 16, 16)]
                             + w01 * gball[l, i, p, pl.ds(32 + h * 16, 16)]
                             + w10 * gball[l, i, p, pl.ds(64 + h * 16, 16)]
                             + w11 * gball[l, i, p, pl.ds(96 + h * 16, 16)])
                        acc[h] = r if i == 0 else acc[h] * r
                for h in range(2):
                    outb[par, p, pl.ds(l * 32 + h * 16, 16)] = acc[h]
            return c

        lax.fori_loop(0, C // 16, grp_body, 0)

    def fire_out(g, par):
        pltpu.async_copy(
            outb.at[par], out.at[pl.ds(wbase + g * C, C)], semo)

    def wait_out():
        pltpu.make_async_copy(
            outb.at[0], out.at[pl.ds(wbase, C)], semo).wait()

    fire_x(0, 0)
    fire_x(1, 1)
    wait_x(0)
    comp_idx(0)
    fire_item(0, 0, 0)
    fire_item(1, 0, 1)

    def chunk_body(g, carry):
        par = lax.rem(g, 2)
        nxt = 1 - par

        @pl.when(g + 1 < NCHUNK)
        def _():
            wait_x(nxt)
            comp_idx(nxt)

        @pl.when(g + 2 < NCHUNK)
        def _():
            fire_x(g + 2, par)

        fire_item(2, par, 2)

        @pl.when(g >= 2)
        def _():
            wait_out()
        wait_item(0)
        combine(0, par)

        @pl.when(g + 1 < NCHUNK)
        def _():
            fire_item(0, nxt, 0)
        wait_item(1)
        combine(1, par)

        @pl.when(g + 1 < NCHUNK)
        def _():
            fire_item(1, nxt, 1)
        wait_item(2)
        combine(2, par)

        fire_out(g, par)
        return carry

    lax.fori_loop(0, NCHUNK, chunk_body, 0)
    wait_out()
    wait_out()


def kernel(x, low_0, low_1, low_2, mid_0, mid_1, mid_2, high_0, high_1, high_2):
    planes = [low_0, low_1, low_2, mid_0, mid_1, mid_2, high_0, high_1, high_2]
    tabs = [_quad_table(p, RES[j // 3]) for j, p in enumerate(planes)]
    xT = x.T

    mesh = plsc.VectorSubcoreMesh(
        core_axis_name="c", subcore_axis_name="s",
        num_cores=NC, num_subcores=NS)
    scratch = [
        pltpu.VMEM((2, 3, C), jnp.float32),
        pltpu.VMEM((2, 3, 3, C), jnp.float32),
        pltpu.VMEM((2, 9, C), jnp.int32),
        pltpu.VMEM((3, 3, C, ROW), jnp.float32),
        pltpu.VMEM((2, C, 96), jnp.float32),
        pltpu.SemaphoreType.DMA,
        pltpu.SemaphoreType.DMA,
        pltpu.SemaphoreType.DMA,
    ]
    run = pl.kernel(
        _sc_body,
        out_type=jax.ShapeDtypeStruct((B, 96), jnp.float32),
        mesh=mesh,
        scratch_types=scratch,
    )
    return run(xT, *tabs)


# speedup vs baseline: 1.4219x; 1.4219x over previous
import jax
import jax.numpy as jnp
from jax import lax
from jax.experimental import pallas as pl
from jax.experimental.pallas import tpu as pltpu
from jax.experimental.pallas import tpu_sc as plsc

F = 32
B = 524288
RES = (64, 128, 256)
DIMIDS = ((0, 1), (0, 2), (1, 2))
NC, NS = 2, 16
NW = NC * NS
PPW = B // NW
C = 128
NCHUNK = PPW // C
ROW = 4 * F


def _quad_table(plane, R):
    t = plane.transpose(1, 2, 0).reshape(R * R, F)
    n = R * R
    tp = jnp.pad(t, ((0, R + 1), (0, 0)))
    return jnp.concatenate(
        [t, tp[1:n + 1], tp[R:n + R], tp[R + 1:n + R + 1]], axis=1)


def _sc_body(x0, x1, x2, t0, t1, t2, t3, t4, t5, t6, t7, t8, out,
             xv, wb, idxb, gball, outb, semx, semg, semo):
    tabs = (t0, t1, t2, t3, t4, t5, t6, t7, t8)
    xs = (x0, x1, x2)
    wid = lax.axis_index("s") * NC + lax.axis_index("c")
    wbase = wid * PPW

    def fire_x(g, par):
        for d in range(3):
            pltpu.async_copy(
                xs[d].at[pl.ds(wbase + g * C, C)], xv.at[par, d], semx)

    def wait_x(par):
        for d in range(3):
            pltpu.make_async_copy(
                xs[d].at[pl.ds(wbase, C)], xv.at[par, d], semx).wait()

    def comp_idx(par):
        for q in range(C // 16):
            sl = pl.ds(q * 16, 16)
            xd = [xv[par, d, sl] for d in range(3)]
            for l, R in enumerate(RES):
                s = (R - 1) * 0.5
                iv = []
                for d in range(3):
                    gco = (xd[d] + 1.0) * s
                    ic = jnp.maximum(
                        jnp.minimum(gco.astype(jnp.int32), R - 2), 0)
                    wb[par, l, d, sl] = gco - ic.astype(jnp.float32)
                    iv.append(ic)
                idxb[par, 3 * l + 0, sl] = iv[1] * R + iv[0]
                idxb[par, 3 * l + 1, sl] = iv[2] * R + iv[0]
                idxb[par, 3 * l + 2, sl] = iv[2] * R + iv[1]

    def fire_item(l2, par2, slot):
        for i in range(3):
            pltpu.async_copy(
                tabs[3 * l2 + i].at[idxb.at[par2, 3 * l2 + i]],
                gball.at[slot, i], semg)

    def wait_item(slot):
        for i in range(3):
            pltpu.make_async_copy(
                tabs[0].at[idxb.at[0, 0]], gball.at[slot, i], semg).wait()

    def combine(l, slot, par):
        def grp_body(q, c):
            wv = [wb[par, l, d, pl.ds(q * 16, 16)] for d in range(3)]
            cw = []
            for a, b in DIMIDS:
                wx, wy = wv[a], wv[b]
                u = 1.0 - wx
                v = 1.0 - wy
                cw.append((u * v, wx * v, u * wy, wx * wy))
            for p16 in range(16):
                p = q * 16 + p16
                acc = [None, None]
                for i in range(3):
                    w00 = cw[i][0][p16]
                    w01 = cw[i][1][p16]
                    w10 = cw[i][2][p16]
                    w11 = cw[i][3][p16]
                    for h in range(2):
                        r = (w00 * gball[slot, i, p, pl.ds(h * 16, 16)]
                             + w01 * gball[slot, i, p, pl.ds(32 + h * 16, 16)]
                             + w10 * gball[slot, i, p, pl.ds(64 + h * 16, 16)]
                             + w11 * gball[slot, i, p, pl.ds(96 + h * 16, 16)])
                        acc[h] = r if i == 0 else acc[h] * r
                for h in range(2):
                    outb[p, pl.ds(l * 32 + h * 16, 16)] = acc[h]
            return c

        lax.fori_loop(0, C // 16, grp_body, 0)

    def fire_out(g):
        pltpu.async_copy(
            outb, out.at[pl.ds(wbase + g * C, C)], semo)

    def wait_out():
        pltpu.make_async_copy(
            outb, out.at[pl.ds(wbase, C)], semo).wait()

    fire_x(0, 0)
    fire_x(1, 1)
    wait_x(0)
    comp_idx(0)
    fire_item(0, 0, 0)

    def outer_body(go, carry):
        for b in range(2):
            g = 2 * go + b

            @pl.when(g + 1 < NCHUNK)
            def _(b=b):
                wait_x(1 - b)
                comp_idx(1 - b)

            @pl.when(g + 2 < NCHUNK)
            def _(g=g, b=b):
                fire_x(g + 2, b)

            for l in range(3):
                if l < 2:
                    fire_item(l + 1, b, (b + l + 1) % 2)
                else:
                    @pl.when(g + 1 < NCHUNK)
                    def _(b=b):
                        fire_item(0, 1 - b, (b + 1) % 2)
                if l == 0:
                    if b == 0:
                        @pl.when(go >= 1)
                        def _():
                            wait_out()
                    else:
                        wait_out()
                wait_item((b + l) % 2)
                combine(l, (b + l) % 2, b)

            fire_out(g)
        return carry

    lax.fori_loop(0, NCHUNK // 2, outer_body, 0)
    wait_out()


def kernel(x, low_0, low_1, low_2, mid_0, mid_1, mid_2, high_0, high_1, high_2):
    planes = [low_0, low_1, low_2, mid_0, mid_1, mid_2, high_0, high_1, high_2]
    tabs = [_quad_table(p, RES[j // 3]) for j, p in enumerate(planes)]
    xs = [x[:, 0], x[:, 1], x[:, 2]]

    mesh = plsc.VectorSubcoreMesh(
        core_axis_name="c", subcore_axis_name="s",
        num_cores=NC, num_subcores=NS)
    scratch = [
        pltpu.VMEM((2, 3, C), jnp.float32),
        pltpu.VMEM((2, 3, 3, C), jnp.float32),
        pltpu.VMEM((2, 9, C), jnp.int32),
        pltpu.VMEM((2, 3, C, ROW), jnp.float32),
        pltpu.VMEM((C, 96), jnp.float32),
        pltpu.SemaphoreType.DMA,
        pltpu.SemaphoreType.DMA,
        pltpu.SemaphoreType.DMA,
    ]
    run = pl.kernel(
        _sc_body,
        out_type=jax.ShapeDtypeStruct((B, 96), jnp.float32),
        mesh=mesh,
        scratch_types=scratch,
    )
    return run(*xs, *tabs)
